# baseline (speedup 1.0000x reference)
import functools

import jax
import jax.numpy as jnp
from jax import lax
from jax.experimental import pallas as pl
from jax.experimental.pallas import tpu as pltpu
from jax.experimental.pallas import tpu_sc as plsc

N = 4096
D = 256
LAYERS = 3
K = 32
ALPHA = 3.0

EMB_BLOCK = 512
ROW_BLOCK = 256
CHUNK = 128
NCHUNK = N // CHUNK

_DN_NT = (((1,), (1,)), ((), ()))
_DN_NN = (((1,), (0,)), ((), ()))


def _embed_body(scale_ref, w1_ref, b1_ref, w2_ref, b2_ref, e1_ref, e2_ref,
                o1_ref, o2_ref):
    x1 = e1_ref[...]
    x2 = e2_ref[...]
    for l in range(LAYERS):
        s = scale_ref[l, 0]
        w1 = w1_ref[l]
        w2 = w2_ref[l]
        b1 = b1_ref[l]
        b2 = b2_ref[l]
        x1 = jnp.tanh(ALPHA * (
            jax.lax.dot_general(x1 * s, w1, _DN_NT,
                                preferred_element_type=jnp.float32) + b1))
        x2 = jnp.tanh(ALPHA * (
            jax.lax.dot_general(x2 * s, w2, _DN_NT,
                                preferred_element_type=jnp.float32) + b2))
        o1_ref[l] = x1
        o2_ref[l] = x2


def _layer_body(v1_ref, v2_ref, v1b_ref, v2b_ref, out_ref):
    v1 = v1_ref[0]
    v2 = v2_ref[0]
    v1b = v1b_ref[0]
    v2b = v2b_ref[0]

    m1 = jax.lax.dot_general(v1b, v2, _DN_NT, preferred_element_type=jnp.float32)
    m2 = jax.lax.dot_general(v2b, v1, _DN_NT, preferred_element_type=jnp.float32)
    adj0 = jnp.maximum(jnp.tanh(ALPHA * (m1 - m2)), 0.0)

    u = jax.lax.bitcast_convert_type(adj0, jnp.int32)
    m = jnp.max(u, axis=1, keepdims=True)
    c_eq_max = jnp.sum((u == m).astype(jnp.int32), axis=1, keepdims=True)
    all_fast = jnp.all(c_eq_max >= K)

    def _search():
        def body(_, carry):
            lo, hi = carry
            mid = (lo + hi + 1) // 2
            cnt = jnp.sum((u >= mid).astype(jnp.int32), axis=1, keepdims=True)
            ok = cnt >= K
            return jnp.where(ok, mid, lo), jnp.where(ok, hi, mid - 1)
        lo0 = jnp.zeros_like(m)
        lo, _ = jax.lax.fori_loop(0, 30, body, (lo0, m))
        return lo

    t = jax.lax.cond(all_fast, lambda: m, _search)
    cnt_gt = jnp.sum((u > t).astype(jnp.int32), axis=1, keepdims=True)
    rrem = (K - cnt_gt).astype(jnp.float32)

    row_i = jax.lax.broadcasted_iota(jnp.int32, (CHUNK, CHUNK), 0)
    col_j = jax.lax.broadcasted_iota(jnp.int32, (CHUNK, CHUNK), 1)
    tri = (row_i <= col_j).astype(jnp.bfloat16)

    carry = jnp.zeros((ROW_BLOCK, 1), jnp.float32)
    for c in range(NCHUNK):
        sl = slice(c * CHUNK, (c + 1) * CHUNK)
        u_c = u[:, sl]
        eq_c = u_c == t
        within = jax.lax.dot_general(eq_c.astype(jnp.bfloat16), tri, _DN_NN,
                                     preferred_element_type=jnp.float32)
        cum = within + carry
        carry = cum[:, CHUNK - 1:CHUNK]
        keep = (u_c > t) | (eq_c & (cum <= rrem))
        out_ref[:, sl] = jnp.where(keep, adj0[:, sl], 0.0)


def _sc_row_scan(adj):
    info = plsc.get_sparse_core_info()
    nc, ns, nl = info.num_cores, info.num_subcores, info.num_lanes
    nw = nc * ns
    rows_per = N // nw

    @functools.partial(
        pl.kernel,
        mesh=plsc.VectorSubcoreMesh(core_axis_name="c", subcore_axis_name="s"),
        out_type=[jax.ShapeDtypeStruct((N,), jnp.float32),
                  jax.ShapeDtypeStruct((N,), jnp.float32)],
        scratch_types=[pltpu.VMEM((N,), jnp.float32),
                       pltpu.VMEM((rows_per,), jnp.float32),
                       pltpu.VMEM((rows_per,), jnp.float32)],
    )
    def k(adj_hbm, t_hbm, c_hbm, row_v, tbuf, cbuf):
        wid = lax.axis_index("s") * nc + lax.axis_index("c")
        base = wid * rows_per
        lane = lax.broadcasted_iota(jnp.int32, (nl,), 0)

        def _splat_max(x):
            return jnp.maximum(plsc.cummax(x),
                               lax.rev(plsc.cummax(lax.rev(x, (0,))), (0,)))

        def _splat_sum(x):
            return plsc.cumsum(x) + lax.rev(plsc.cumsum(lax.rev(x, (0,))), (0,)) - x

        def grp_body(g, carry):
            def row_body(j, accs):
                tacc, cacc = accs
                pltpu.sync_copy(adj_hbm.at[base + g * nl + j], row_v)

                def mx_body(q, acc):
                    return jnp.maximum(acc, row_v[pl.ds(q * nl, nl)])
                mvec = lax.fori_loop(0, N // nl, mx_body,
                                     jnp.zeros((nl,), jnp.float32))
                msplat = mvec

                def ct_body(q, acc):
                    v = row_v[pl.ds(q * nl, nl)]
                    return acc + jnp.where(v == msplat, 1.0, 0.0)
                csplat = lax.fori_loop(0, N // nl, ct_body,
                                       jnp.zeros((nl,), jnp.float32))
                sel = lane == j
                return (jnp.where(sel, msplat, tacc),
                        jnp.where(sel, csplat, cacc))

            tacc, cacc = lax.fori_loop(
                0, nl, row_body,
                (jnp.zeros((nl,), jnp.float32), jnp.zeros((nl,), jnp.float32)))
            tbuf[pl.ds(g * nl, nl)] = tacc
            cbuf[pl.ds(g * nl, nl)] = cacc
            return carry

        lax.fori_loop(0, rows_per // nl, grp_body, 0)
        pltpu.sync_copy(tbuf, t_hbm.at[pl.ds(base, rows_per)])
        pltpu.sync_copy(cbuf, c_hbm.at[pl.ds(base, rows_per)])

    return k(adj)


@jax.jit
def kernel(idx, scale_idx, scale_set, emb1, emb2, W1, b1, W2, b2):
    del idx, scale_idx

    v1s, v2s = pl.pallas_call(
        _embed_body,
        grid=(N // EMB_BLOCK,),
        in_specs=[
            pl.BlockSpec((LAYERS, 1), lambda r: (0, 0)),
            pl.BlockSpec((LAYERS, D, D), lambda r: (0, 0, 0)),
            pl.BlockSpec((LAYERS, D), lambda r: (0, 0)),
            pl.BlockSpec((LAYERS, D, D), lambda r: (0, 0, 0)),
            pl.BlockSpec((LAYERS, D), lambda r: (0, 0)),
            pl.BlockSpec((EMB_BLOCK, D), lambda r: (r, 0)),
            pl.BlockSpec((EMB_BLOCK, D), lambda r: (r, 0)),
        ],
        out_specs=[
            pl.BlockSpec((LAYERS, EMB_BLOCK, D), lambda r: (0, r, 0)),
            pl.BlockSpec((LAYERS, EMB_BLOCK, D), lambda r: (0, r, 0)),
        ],
        out_shape=[
            jax.ShapeDtypeStruct((LAYERS, N, D), jnp.float32),
            jax.ShapeDtypeStruct((LAYERS, N, D), jnp.float32),
        ],
    )(scale_set.reshape(LAYERS, 1), W1, b1, W2, b2, emb1, emb2)

    outs = []
    for l in range(LAYERS):
        adj = pl.pallas_call(
            functools.partial(_layer_body),
            grid=(N // ROW_BLOCK,),
            in_specs=[
                pl.BlockSpec((1, N, D), lambda r, _l=l: (_l, 0, 0)),
                pl.BlockSpec((1, N, D), lambda r, _l=l: (_l, 0, 0)),
                pl.BlockSpec((1, ROW_BLOCK, D), lambda r, _l=l: (_l, r, 0)),
                pl.BlockSpec((1, ROW_BLOCK, D), lambda r, _l=l: (_l, r, 0)),
            ],
            out_specs=pl.BlockSpec((ROW_BLOCK, N), lambda r: (r, 0)),
            out_shape=jax.ShapeDtypeStruct((N, N), jnp.float32),
        )(v1s, v2s, v1s, v2s)
        sc_t, sc_c = _sc_row_scan(adj)
        adj = adj.at[0, 0].add((jnp.sum(sc_t) + jnp.sum(sc_c)) * 0.0)
        outs.append(adj)
    return tuple(outs)


# speedup vs baseline: 1.2923x; 1.2923x over previous
import functools

import jax
import jax.numpy as jnp
from jax.experimental import pallas as pl
from jax.experimental.pallas import tpu as pltpu

N = 4096
D = 256
LAYERS = 3
K = 32
ALPHA = 3.0

EMB_BLOCK = 512
ROW_BLOCK = 256
CHUNK = 128
NCHUNK = N // CHUNK

_DN_NT = (((1,), (1,)), ((), ()))
_DN_NN = (((1,), (0,)), ((), ()))


def _embed_body(scale_ref, w1_ref, b1_ref, w2_ref, b2_ref, e1_ref, e2_ref,
                o1_ref, o2_ref):
    x1 = e1_ref[...]
    x2 = e2_ref[...]
    for l in range(LAYERS):
        s = scale_ref[l, 0]
        w1 = w1_ref[l]
        w2 = w2_ref[l]
        b1 = b1_ref[l]
        b2 = b2_ref[l]
        x1 = jnp.tanh(ALPHA * (
            jax.lax.dot_general(x1 * s, w1, _DN_NT,
                                preferred_element_type=jnp.float32) + b1))
        x2 = jnp.tanh(ALPHA * (
            jax.lax.dot_general(x2 * s, w2, _DN_NT,
                                preferred_element_type=jnp.float32) + b2))
        o1_ref[l] = x1
        o2_ref[l] = x2


def _layer_body(v1_ref, v2_ref, v1b_ref, v2b_ref, out_ref):
    v1 = v1_ref[0]
    v2 = v2_ref[0]
    v1b = v1b_ref[0]
    v2b = v2b_ref[0]

    m1 = jax.lax.dot_general(v1b, v2, _DN_NT, preferred_element_type=jnp.float32)
    m2 = jax.lax.dot_general(v2b, v1, _DN_NT, preferred_element_type=jnp.float32)
    adj0 = jnp.maximum(jnp.tanh(ALPHA * (m1 - m2)), 0.0)

    u = jax.lax.bitcast_convert_type(adj0, jnp.int32)
    m = jnp.max(u, axis=1, keepdims=True)
    all_fast = jnp.all(m == 0)

    def _search():
        def body(_, carry):
            lo, hi = carry
            mid = (lo + hi + 1) // 2
            cnt = jnp.sum((u >= mid).astype(jnp.int32), axis=1, keepdims=True)
            ok = cnt >= K
            return jnp.where(ok, mid, lo), jnp.where(ok, hi, mid - 1)
        lo0 = jnp.zeros_like(m)
        lo, _ = jax.lax.fori_loop(0, 30, body, (lo0, m))
        return lo

    t = jax.lax.cond(all_fast, lambda: m, _search)
    cnt_gt = jnp.sum((u > t).astype(jnp.int32), axis=1, keepdims=True)
    rrem = (K - cnt_gt).astype(jnp.float32)

    row_i = jax.lax.broadcasted_iota(jnp.int32, (CHUNK, CHUNK), 0)
    col_j = jax.lax.broadcasted_iota(jnp.int32, (CHUNK, CHUNK), 1)
    tri = (row_i <= col_j).astype(jnp.bfloat16)

    carry = jnp.zeros((ROW_BLOCK, 1), jnp.float32)
    for c in range(NCHUNK):
        sl = slice(c * CHUNK, (c + 1) * CHUNK)
        u_c = u[:, sl]
        eq_c = u_c == t
        within = jax.lax.dot_general(eq_c.astype(jnp.bfloat16), tri, _DN_NN,
                                     preferred_element_type=jnp.float32)
        cum = within + carry
        carry = cum[:, CHUNK - 1:CHUNK]
        keep = (u_c > t) | (eq_c & (cum <= rrem))
        out_ref[:, sl] = jnp.where(keep, adj0[:, sl], 0.0)


@jax.jit
def kernel(idx, scale_idx, scale_set, emb1, emb2, W1, b1, W2, b2):
    del idx, scale_idx

    v1s, v2s = pl.pallas_call(
        _embed_body,
        grid=(N // EMB_BLOCK,),
        in_specs=[
            pl.BlockSpec((LAYERS, 1), lambda r: (0, 0)),
            pl.BlockSpec((LAYERS, D, D), lambda r: (0, 0, 0)),
            pl.BlockSpec((LAYERS, D), lambda r: (0, 0)),
            pl.BlockSpec((LAYERS, D, D), lambda r: (0, 0, 0)),
            pl.BlockSpec((LAYERS, D), lambda r: (0, 0)),
            pl.BlockSpec((EMB_BLOCK, D), lambda r: (r, 0)),
            pl.BlockSpec((EMB_BLOCK, D), lambda r: (r, 0)),
        ],
        out_specs=[
            pl.BlockSpec((LAYERS, EMB_BLOCK, D), lambda r: (0, r, 0)),
            pl.BlockSpec((LAYERS, EMB_BLOCK, D), lambda r: (0, r, 0)),
        ],
        out_shape=[
            jax.ShapeDtypeStruct((LAYERS, N, D), jnp.float32),
            jax.ShapeDtypeStruct((LAYERS, N, D), jnp.float32),
        ],
    )(scale_set.reshape(LAYERS, 1), W1, b1, W2, b2, emb1, emb2)

    outs = []
    for l in range(LAYERS):
        adj = pl.pallas_call(
            functools.partial(_layer_body),
            grid=(N // ROW_BLOCK,),
            in_specs=[
                pl.BlockSpec((1, N, D), lambda r, _l=l: (_l, 0, 0)),
                pl.BlockSpec((1, N, D), lambda r, _l=l: (_l, 0, 0)),
                pl.BlockSpec((1, ROW_BLOCK, D), lambda r, _l=l: (_l, r, 0)),
                pl.BlockSpec((1, ROW_BLOCK, D), lambda r, _l=l: (_l, r, 0)),
            ],
            out_specs=pl.BlockSpec((ROW_BLOCK, N), lambda r: (r, 0)),
            out_shape=jax.ShapeDtypeStruct((N, N), jnp.float32),
        )(v1s, v2s, v1s, v2s)
        outs.append(adj)
    return tuple(outs)


# speedup vs baseline: 1.6892x; 1.3071x over previous
import functools

import jax
import jax.numpy as jnp
from jax.experimental import pallas as pl
from jax.experimental.pallas import tpu as pltpu

N = 4096
D = 256
LAYERS = 3
K = 32
ALPHA = 3.0

EMB_BLOCK = 512
ROW_BLOCK = 256
CHUNK = 128
NCHUNK = N // CHUNK

_DN_NT = (((1,), (1,)), ((), ()))
_DN_NN = (((1,), (0,)), ((), ()))


def _embed_body(scale_ref, w1_ref, b1_ref, w2_ref, b2_ref, e1_ref, e2_ref,
                o1_ref, o2_ref):
    x1 = e1_ref[...]
    x2 = e2_ref[...]
    for l in range(LAYERS):
        s = scale_ref[l, 0]
        w1 = w1_ref[l]
        w2 = w2_ref[l]
        b1 = b1_ref[l]
        b2 = b2_ref[l]
        x1 = jnp.tanh(ALPHA * (
            jax.lax.dot_general(x1 * s, w1, _DN_NT,
                                preferred_element_type=jnp.float32) + b1))
        x2 = jnp.tanh(ALPHA * (
            jax.lax.dot_general(x2 * s, w2, _DN_NT,
                                preferred_element_type=jnp.float32) + b2))
        o1_ref[l] = x1
        o2_ref[l] = x2


def _layer_body(v1_ref, v2_ref, v1b_ref, v2b_ref, out_ref):
    v1 = v1_ref[0]
    v2 = v2_ref[0]
    v1b = v1b_ref[0]
    v2b = v2b_ref[0]

    m1 = jax.lax.dot_general(v1b, v2, _DN_NT, preferred_element_type=jnp.float32)
    m2 = jax.lax.dot_general(v2b, v1, _DN_NT, preferred_element_type=jnp.float32)
    adj0 = jnp.maximum(jnp.tanh(ALPHA * (m1 - m2)), 0.0)

    u = jax.lax.bitcast_convert_type(adj0, jnp.int32)
    m = jnp.max(u, axis=1, keepdims=True)
    c_eq_max = jnp.sum((u == m).astype(jnp.int32), axis=1, keepdims=True)
    all_fast = jnp.all(c_eq_max >= K)

    def _search():
        def body(_, carry):
            lo, hi = carry
            mid = (lo + hi + 1) // 2
            cnt = jnp.sum((u >= mid).astype(jnp.int32), axis=1, keepdims=True)
            ok = cnt >= K
            return jnp.where(ok, mid, lo), jnp.where(ok, hi, mid - 1)
        lo0 = jnp.zeros_like(m)
        lo, _ = jax.lax.fori_loop(0, 30, body, (lo0, m))
        return lo

    t = jax.lax.cond(all_fast, lambda: m, _search)
    cnt_gt = jnp.sum((u > t).astype(jnp.int32), axis=1, keepdims=True)
    rrem = (K - cnt_gt).astype(jnp.float32)

    row_i = jax.lax.broadcasted_iota(jnp.int32, (CHUNK, CHUNK), 0)
    col_j = jax.lax.broadcasted_iota(jnp.int32, (CHUNK, CHUNK), 1)
    tri = (row_i <= col_j).astype(jnp.bfloat16)

    carry = jnp.zeros((ROW_BLOCK, 1), jnp.float32)
    for c in range(NCHUNK):
        sl = slice(c * CHUNK, (c + 1) * CHUNK)
        u_c = u[:, sl]
        eq_c = u_c == t
        within = jax.lax.dot_general(eq_c.astype(jnp.bfloat16), tri, _DN_NN,
                                     preferred_element_type=jnp.float32)
        cum = within + carry
        carry = cum[:, CHUNK - 1:CHUNK]
        keep = (u_c > t) | (eq_c & (cum <= rrem))
        out_ref[:, sl] = jnp.where(keep, adj0[:, sl], 0.0)


@jax.jit
def kernel(idx, scale_idx, scale_set, emb1, emb2, W1, b1, W2, b2):
    del idx, scale_idx

    v1s, v2s = pl.pallas_call(
        _embed_body,
        grid=(N // EMB_BLOCK,),
        in_specs=[
            pl.BlockSpec((LAYERS, 1), lambda r: (0, 0)),
            pl.BlockSpec((LAYERS, D, D), lambda r: (0, 0, 0)),
            pl.BlockSpec((LAYERS, D), lambda r: (0, 0)),
            pl.BlockSpec((LAYERS, D, D), lambda r: (0, 0, 0)),
            pl.BlockSpec((LAYERS, D), lambda r: (0, 0)),
            pl.BlockSpec((EMB_BLOCK, D), lambda r: (r, 0)),
            pl.BlockSpec((EMB_BLOCK, D), lambda r: (r, 0)),
        ],
        out_specs=[
            pl.BlockSpec((LAYERS, EMB_BLOCK, D), lambda r: (0, r, 0)),
            pl.BlockSpec((LAYERS, EMB_BLOCK, D), lambda r: (0, r, 0)),
        ],
        out_shape=[
            jax.ShapeDtypeStruct((LAYERS, N, D), jnp.float32),
            jax.ShapeDtypeStruct((LAYERS, N, D), jnp.float32),
        ],
    )(scale_set.reshape(LAYERS, 1), W1, b1, W2, b2, emb1, emb2)

    outs = []
    for l in range(LAYERS):
        adj = pl.pallas_call(
            functools.partial(_layer_body),
            grid=(N // ROW_BLOCK,),
            in_specs=[
                pl.BlockSpec((1, N, D), lambda r, _l=l: (_l, 0, 0)),
                pl.BlockSpec((1, N, D), lambda r, _l=l: (_l, 0, 0)),
                pl.BlockSpec((1, ROW_BLOCK, D), lambda r, _l=l: (_l, r, 0)),
                pl.BlockSpec((1, ROW_BLOCK, D), lambda r, _l=l: (_l, r, 0)),
            ],
            out_specs=pl.BlockSpec((ROW_BLOCK, N), lambda r: (r, 0)),
            out_shape=jax.ShapeDtypeStruct((N, N), jnp.float32),
        )(v1s, v2s, v1s, v2s)
        outs.append(adj)
    return tuple(outs)
